# Initial kernel scaffold; baseline (speedup 1.0000x reference)
import jax
import jax.numpy as jnp
from jax.experimental import pallas as pl


def kernel(x, edge_index, batch, W1, a_s1, a_d1, b1, W2, a_s2, a_d2, b2, W3, a_s3, a_d3, b3, lin1_w, lin1_b, lin2_w, lin2_b):
    raise NotImplementedError("write your pallas kernel here")


# speedup vs baseline: 1.1375x; 1.1375x over previous
import jax
import jax.numpy as jnp
from jax.experimental import pallas as pl
from jax.experimental.pallas import tpu as pltpu

N = 10000
HEADS = 8
NHID = 16
NGRAPH = 64
NCLS = 10
F = HEADS * NHID


def _gat_conv(x, src, dst, W, a_s, a_d, b, heads, ch):
    n = x.shape[0]
    h = (x @ W).reshape(n, heads, ch)
    al = (h * a_s[None, :, :]).sum(-1)
    ar = (h * a_d[None, :, :]).sum(-1)
    e = al[src] + ar[dst]
    e = jnp.where(e > 0, e, 0.2 * e)
    ex = jnp.exp(e)
    es = jax.ops.segment_sum(ex, dst, num_segments=n)
    alpha = ex / (es[dst] + 1e-16)
    msg = h[src] * alpha[:, :, None]
    out = jax.ops.segment_sum(msg, dst, num_segments=n)
    return out.reshape(n, heads * ch) + b


def _final_kernel(h_ref, batch_ref, lin1w_ref, lin1b_ref, lin2w_ref, lin2b_ref, out_ref):
    h = h_ref[...]
    batch = batch_ref[...]
    onehot = (batch[:, None] == jax.lax.broadcasted_iota(jnp.int32, (1, NGRAPH), 1)).astype(jnp.float32)
    sums = jnp.dot(onehot.T, h, preferred_element_type=jnp.float32)
    cnt = jnp.sum(onehot, axis=0)
    pooled = sums / jnp.maximum(cnt, 1.0)[:, None]
    z = jnp.maximum(jnp.dot(pooled, lin1w_ref[...], preferred_element_type=jnp.float32) + lin1b_ref[...], 0.0)
    z = jnp.dot(z, lin2w_ref[...], preferred_element_type=jnp.float32) + lin2b_ref[...]
    z = z - jnp.max(z, axis=-1, keepdims=True)
    out_ref[...] = z - jnp.log(jnp.sum(jnp.exp(z), axis=-1, keepdims=True))


def kernel(x, edge_index, batch, W1, a_s1, a_d1, b1, W2, a_s2, a_d2, b2, W3, a_s3, a_d3, b3, lin1_w, lin1_b, lin2_w, lin2_b):
    n = x.shape[0]
    loop = jnp.arange(n, dtype=edge_index.dtype)
    src = jnp.concatenate([edge_index[0], loop])
    dst = jnp.concatenate([edge_index[1], loop])
    h = jax.nn.elu(_gat_conv(x, src, dst, W1, a_s1, a_d1, b1, HEADS, NHID))
    h = jax.nn.elu(_gat_conv(h, src, dst, W2, a_s2, a_d2, b2, 1, HEADS * NHID))
    h = jax.nn.elu(_gat_conv(h, src, dst, W3, a_s3, a_d3, b3, 1, HEADS * NHID))
    out = pl.pallas_call(
        _final_kernel,
        out_shape=jax.ShapeDtypeStruct((NGRAPH, NCLS), jnp.float32),
    )(h, batch, lin1_w, lin1_b, lin2_w, lin2_b)
    return out


# speedup vs baseline: 30.9308x; 27.1930x over previous
import jax
import jax.numpy as jnp
from jax import lax
from jax.experimental import pallas as pl
from jax.experimental.pallas import tpu as pltpu
from jax.experimental.pallas import tpu_sc as plsc

N = 10000
E = 320000
F = 128
NH = 8
NGRAPH = 64
NCLS = 10
NTILES = 32
EPT = E // NTILES
RSTEP = 624
RSPAN = 640
BLK = 1000
GRID = N // BLK

_f32 = jnp.float32
_i32 = jnp.int32


def _mesh():
    return plsc.VectorSubcoreMesh(core_axis_name="c", subcore_axis_name="s",
                                  num_cores=2, num_subcores=16)


def _leaky_exp(a, b):
    e = a + b
    return jnp.exp(jnp.where(e > 0, e, 0.2 * e))


def _zero_m(m_sh, rows_v, rbase, cb):
    zero16 = jnp.zeros((16,), _f32)

    @pl.loop(0, cb)
    def _(e):
        for kk in range(8):
            rows_v[e, pl.ds(16 * kk, 16)] = zero16

    for i in range(RSPAN // cb):
        pltpu.sync_copy(rows_v, m_sh.at[pl.ds(rbase + i * cb, cb)])


def _export_m(m_sh, m_out, c, rbase, cb):
    for i in range(RSPAN // cb):
        pltpu.sync_copy(m_sh.at[pl.ds(rbase + i * cb, cb)],
                        m_out.at[c, pl.ds(rbase + i * cb, cb), :])


def _make_sc_layer8():
    CA = 400
    CB = 80

    def body(h_hbm, alT_hbm, arT_hbm, src_hbm, dst_hbm,
             m_out, es_out, w_out,
             m_sh, al_v, ar_v, es_v, srcA_v, dstA_v, wa_v,
             srcB_v, dstB_v, wb0_v, wb1_v, wb2_v, wb3_v,
             wb4_v, wb5_v, wb6_v, wb7_v, rows_v, sem):
        c = lax.axis_index("c")
        s = lax.axis_index("s")
        wid = s * 2 + c
        ebase = wid * EPT
        rbase = s * RSTEP
        zero16 = jnp.zeros((16,), _f32)
        wbs = (wb0_v, wb1_v, wb2_v, wb3_v, wb4_v, wb5_v, wb6_v, wb7_v)

        _zero_m(m_sh, rows_v, rbase, CB)
        plsc.subcore_barrier()

        for g in range(NH):
            pltpu.sync_copy(alT_hbm.at[pl.ds(g * N, N)], al_v)
            pltpu.sync_copy(arT_hbm.at[pl.ds(g * N, N)], ar_v)

            @pl.loop(0, N // 16)
            def _(i):
                es_v[pl.ds(16 * i, 16)] = zero16

            @pl.loop(0, EPT // CA)
            def _(ci):
                cb = ebase + ci * CA
                pltpu.sync_copy(src_hbm.at[pl.ds(cb, CA)], srcA_v)
                pltpu.sync_copy(dst_hbm.at[pl.ds(cb, CA)], dstA_v)

                @pl.loop(0, CA // 16)
                def _(j):
                    s16 = srcA_v[pl.ds(16 * j, 16)]
                    d16 = dstA_v[pl.ds(16 * j, 16)]
                    w = _leaky_exp(plsc.load_gather(al_v, [s16]),
                                   plsc.load_gather(ar_v, [d16]))
                    plsc.addupdate_scatter(es_v, [d16], w)
                    wa_v[pl.ds(16 * j, 16)] = w

                pltpu.sync_copy(wa_v, w_out.at[pl.ds(g * E + cb, CA)])

            pltpu.sync_copy(es_v, es_out.at[pl.ds((wid * NH + g) * N, N)])

        @pl.loop(0, EPT // CB)
        def _(ci):
            cb = ebase + ci * CB
            pltpu.sync_copy(src_hbm.at[pl.ds(cb, CB)], srcB_v)
            pltpu.sync_copy(dst_hbm.at[pl.ds(cb, CB)], dstB_v)
            for g in range(NH):
                pltpu.sync_copy(w_out.at[pl.ds(g * E + cb, CB)], wbs[g])
            pltpu.async_copy(h_hbm.at[srcB_v], rows_v, sem).wait()

            @pl.loop(0, CB // 16)
            def _(q):
                for g in range(NH):
                    wq = wbs[g][pl.ds(16 * q, 16)]
                    for i in range(16):
                        e = 16 * q + i
                        rows_v[e, pl.ds(16 * g, 16)] = (
                            rows_v[e, pl.ds(16 * g, 16)] * wq[i])

            pltpu.sync_copy(rows_v, m_sh.at[dstB_v], add=True)

        plsc.subcore_barrier()
        _export_m(m_sh, m_out, c, rbase, CB)

    return pl.kernel(
        body,
        out_type=[
            jax.ShapeDtypeStruct((2, N, F), _f32),
            jax.ShapeDtypeStruct((NTILES * NH * N,), _f32),
            jax.ShapeDtypeStruct((NH * E,), _f32),
        ],
        mesh=_mesh(),
        compiler_params=pltpu.CompilerParams(needs_layout_passes=False),
        scratch_types=[
            pltpu.VMEM_SHARED((N, F), _f32),
            pltpu.VMEM((N,), _f32),
            pltpu.VMEM((N,), _f32),
            pltpu.VMEM((N,), _f32),
            pltpu.VMEM((CA,), _i32),
            pltpu.VMEM((CA,), _i32),
            pltpu.VMEM((CA,), _f32),
            pltpu.VMEM((CB,), _i32),
            pltpu.VMEM((CB,), _i32),
        ] + [pltpu.VMEM((CB,), _f32) for _ in range(NH)] + [
            pltpu.VMEM((CB, F), _f32),
            pltpu.SemaphoreType.DMA,
        ],
    )


def _make_sc_layer1():
    CB = 80

    def body(h_hbm, al_hbm, ar_hbm, src_hbm, dst_hbm, m_out, es_out,
             m_sh, al_v, ar_v, es_v, src_v, dst_v, w_v, rows_v, sem):
        c = lax.axis_index("c")
        s = lax.axis_index("s")
        wid = s * 2 + c
        ebase = wid * EPT
        rbase = s * RSTEP
        zero16 = jnp.zeros((16,), _f32)

        _zero_m(m_sh, rows_v, rbase, CB)
        plsc.subcore_barrier()

        pltpu.sync_copy(al_hbm, al_v)
        pltpu.sync_copy(ar_hbm, ar_v)

        @pl.loop(0, N // 16)
        def _(i):
            es_v[pl.ds(16 * i, 16)] = zero16

        @pl.loop(0, EPT // CB)
        def _(ci):
            cb = ebase + ci * CB
            pltpu.sync_copy(src_hbm.at[pl.ds(cb, CB)], src_v)
            pltpu.sync_copy(dst_hbm.at[pl.ds(cb, CB)], dst_v)
            pltpu.async_copy(h_hbm.at[src_v], rows_v, sem).wait()

            @pl.loop(0, CB // 16)
            def _(j):
                s16 = src_v[pl.ds(16 * j, 16)]
                d16 = dst_v[pl.ds(16 * j, 16)]
                w = _leaky_exp(plsc.load_gather(al_v, [s16]),
                               plsc.load_gather(ar_v, [d16]))
                w_v[pl.ds(16 * j, 16)] = w
                plsc.addupdate_scatter(es_v, [d16], w)

            @pl.loop(0, CB // 16)
            def _(j):
                wvec = w_v[pl.ds(16 * j, 16)]
                for i in range(16):
                    e = 16 * j + i
                    for kk in range(8):
                        rows_v[e, pl.ds(16 * kk, 16)] = (
                            rows_v[e, pl.ds(16 * kk, 16)] * wvec[i])

            pltpu.sync_copy(rows_v, m_sh.at[dst_v], add=True)

        pltpu.sync_copy(es_v, es_out.at[pl.ds(wid * N, N)])
        plsc.subcore_barrier()
        _export_m(m_sh, m_out, c, rbase, CB)

    return pl.kernel(
        body,
        out_type=[
            jax.ShapeDtypeStruct((2, N, F), _f32),
            jax.ShapeDtypeStruct((NTILES * N,), _f32),
        ],
        mesh=_mesh(),
        compiler_params=pltpu.CompilerParams(needs_layout_passes=False),
        scratch_types=[
            pltpu.VMEM_SHARED((N, F), _f32),
            pltpu.VMEM((N,), _f32),
            pltpu.VMEM((N,), _f32),
            pltpu.VMEM((N,), _f32),
            pltpu.VMEM((CB,), _i32),
            pltpu.VMEM((CB,), _i32),
            pltpu.VMEM((CB,), _f32),
            pltpu.VMEM((CB, F), _f32),
            pltpu.SemaphoreType.DMA,
        ],
    )


_sc_cache = {}


def _sc_layer(H):
    if H not in _sc_cache:
        _sc_cache[H] = _make_sc_layer8() if H == 8 else _make_sc_layer1()
    return _sc_cache[H]


def _proj_kernel(x_ref, w_ref, as_ref, ad_ref, h_ref, al_ref, ar_ref):
    h = jnp.dot(x_ref[...], w_ref[...], preferred_element_type=_f32)
    h_ref[...] = h
    al_ref[...] = jnp.dot(h, as_ref[...], preferred_element_type=_f32)
    ar_ref[...] = jnp.dot(h, ad_ref[...], preferred_element_type=_f32)


def _head_expander(H):
    if H == 1:
        return jnp.ones((1, F), _f32)
    width = F // NH
    return (lax.broadcasted_iota(_i32, (H, H * width), 0)
            == lax.broadcasted_iota(_i32, (H, H * width), 1) // width
            ).astype(_f32)


def _combine_x(H, m_ref, es_ref, al_ref, ar_ref, h_ref, b_ref):
    aa = al_ref[...] + ar_ref[...]
    wself = jnp.exp(jnp.where(aa > 0, aa, 0.2 * aa))
    nt = NTILES * H
    r = (lax.broadcasted_iota(_i32, (nt, H), 0) % H
         == lax.broadcasted_iota(_i32, (nt, H), 1)).astype(_f32)
    es = jnp.dot(es_ref[...], r, preferred_element_type=_f32)
    es = es + wself
    p = _head_expander(H)
    wx = jnp.dot(wself, p, preferred_element_type=_f32)
    esx = jnp.dot(es, p, preferred_element_type=_f32)
    m = m_ref[0] + m_ref[1] + wx * h_ref[...]
    x = m / esx + b_ref[...]
    return jnp.where(x > 0, x, jnp.exp(jnp.minimum(x, 0.0)) - 1.0)


def _make_combine_kernel(H):
    def kern(m_ref, es_ref, al_ref, ar_ref, h_ref, b_ref,
             w_ref, as_ref, ad_ref, hn_ref, aln_ref, arn_ref):
        x = _combine_x(H, m_ref, es_ref, al_ref, ar_ref, h_ref, b_ref)
        hn = jnp.dot(x, w_ref[...], preferred_element_type=_f32)
        hn_ref[...] = hn
        aln_ref[...] = jnp.dot(hn, as_ref[...], preferred_element_type=_f32)
        arn_ref[...] = jnp.dot(hn, ad_ref[...], preferred_element_type=_f32)
    return kern


def _pool_kernel(m_ref, es_ref, al_ref, ar_ref, h_ref, b_ref, batch_ref,
                 sums_ref, cnt_ref):
    x = _combine_x(1, m_ref, es_ref, al_ref, ar_ref, h_ref, b_ref)
    bb = batch_ref[...].reshape(1, BLK)
    onehot = (bb == lax.broadcasted_iota(_i32, (NGRAPH, 1), 0)).astype(_f32)

    @pl.when(pl.program_id(0) == 0)
    def _():
        sums_ref[...] = jnp.zeros_like(sums_ref)
        cnt_ref[...] = jnp.zeros_like(cnt_ref)

    sums_ref[...] += jnp.dot(onehot, x, preferred_element_type=_f32)
    cnt_ref[...] += jnp.dot(onehot, jnp.ones((BLK, F), _f32),
                            preferred_element_type=_f32)


def _mlp_kernel(sums_ref, cnt_ref, l1w_ref, l1b_ref, l2w_ref, l2b_ref, out_ref):
    pooled = sums_ref[...] / jnp.maximum(cnt_ref[...], 1.0)
    z = jnp.dot(pooled, l1w_ref[...], preferred_element_type=_f32) + l1b_ref[...]
    z = jnp.maximum(z, 0.0)
    z = jnp.dot(z, l2w_ref[...], preferred_element_type=_f32) + l2b_ref[...]
    z = z - jnp.max(z, axis=-1, keepdims=True)
    out_ref[...] = z - jnp.log(jnp.sum(jnp.exp(z), axis=-1, keepdims=True))


def _row_spec():
    return pl.BlockSpec((BLK, F), lambda i: (i, 0))


def _head_spec(H):
    return pl.BlockSpec((BLK, H), lambda i: (i, 0))


def _fixed(shape):
    return pl.BlockSpec(shape, lambda i: tuple(0 for _ in shape))


def _proj(x, w, a_s, a_d, H):
    return pl.pallas_call(
        _proj_kernel,
        grid=(GRID,),
        in_specs=[_row_spec(), _fixed((F, F)), _fixed((F, H)), _fixed((F, H))],
        out_specs=[_row_spec(), _head_spec(H), _head_spec(H)],
        out_shape=[
            jax.ShapeDtypeStruct((N, F), _f32),
            jax.ShapeDtypeStruct((N, H), _f32),
            jax.ShapeDtypeStruct((N, H), _f32),
        ],
    )(x, w, a_s, a_d)


def _es_spec(H):
    return pl.BlockSpec((BLK, NTILES * H), lambda i: (i, 0))


def _es_shaped(es, H):
    return es.reshape(NTILES * H, N).T


def _combine_specs(H):
    return [
        pl.BlockSpec((2, BLK, F), lambda i: (0, i, 0)),
        _es_spec(H),
        _head_spec(H),
        _head_spec(H),
        _row_spec(),
        _fixed((1, F)),
    ]


def _combine(H, H2, m, es, al, ar, h, b, w, a_s, a_d):
    return pl.pallas_call(
        _make_combine_kernel(H),
        grid=(GRID,),
        in_specs=_combine_specs(H) + [
            _fixed((F, F)), _fixed((F, H2)), _fixed((F, H2))],
        out_specs=[_row_spec(), _head_spec(H2), _head_spec(H2)],
        out_shape=[
            jax.ShapeDtypeStruct((N, F), _f32),
            jax.ShapeDtypeStruct((N, H2), _f32),
            jax.ShapeDtypeStruct((N, H2), _f32),
        ],
    )(m, _es_shaped(es, H), al, ar, h, b, w, a_s, a_d)


def _pool(m, es, al, ar, h, b, batch3):
    return pl.pallas_call(
        _pool_kernel,
        grid=(GRID,),
        in_specs=_combine_specs(1) + [
            pl.BlockSpec((1, 1, BLK), lambda i: (i, 0, 0))],
        out_specs=[_fixed((NGRAPH, F)), _fixed((NGRAPH, F))],
        out_shape=[
            jax.ShapeDtypeStruct((NGRAPH, F), _f32),
            jax.ShapeDtypeStruct((NGRAPH, F), _f32),
        ],
    )(m, _es_shaped(es, 1), al, ar, h, b, batch3)


def _expand_a(a, heads):
    flat = a.reshape(-1)
    if heads == 8:
        col = jnp.arange(F) // 16
        row = jnp.arange(NH)
        return (col[:, None] == row[None, :]).astype(_f32) * flat[:, None]
    return flat[:, None]


def kernel(x, edge_index, batch, W1, a_s1, a_d1, b1, W2, a_s2, a_d2, b2,
           W3, a_s3, a_d3, b3, lin1_w, lin1_b, lin2_w, lin2_b):
    src = edge_index[0]
    dst = edge_index[1]
    batch3 = batch.reshape(GRID, 1, BLK)

    as1, ad1 = _expand_a(a_s1, 8), _expand_a(a_d1, 8)
    as2, ad2 = _expand_a(a_s2, 1), _expand_a(a_d2, 1)
    as3, ad3 = _expand_a(a_s3, 1), _expand_a(a_d3, 1)
    b1r, b2r, b3r = b1.reshape(1, F), b2.reshape(1, F), b3.reshape(1, F)

    h1, al1, ar1 = _proj(x, W1, as1, ad1, 8)
    alT1 = al1.T.reshape(-1)
    arT1 = ar1.T.reshape(-1)
    m1, es1, _ = _sc_layer(8)(h1, alT1, arT1, src, dst)
    h2, al2, ar2 = _combine(8, 1, m1, es1, al1, ar1, h1, b1r, W2, as2, ad2)
    m2, es2 = _sc_layer(1)(h2, al2.reshape(-1), ar2.reshape(-1), src, dst)
    h3, al3, ar3 = _combine(1, 1, m2, es2, al2, ar2, h2, b2r, W3, as3, ad3)
    m3, es3 = _sc_layer(1)(h3, al3.reshape(-1), ar3.reshape(-1), src, dst)
    sums, cnt = _pool(m3, es3, al3, ar3, h3, b3r, batch3)
    out = pl.pallas_call(
        _mlp_kernel,
        out_shape=jax.ShapeDtypeStruct((NGRAPH, NCLS), _f32),
    )(sums, cnt, lin1_w, lin1_b.reshape(1, -1), lin2_w, lin2_b.reshape(1, -1))
    return out


# speedup vs baseline: 33.3853x; 1.0794x over previous
import jax
import jax.numpy as jnp
from jax import lax
from jax.experimental import pallas as pl
from jax.experimental.pallas import tpu as pltpu
from jax.experimental.pallas import tpu_sc as plsc

N = 10000
E = 320000
F = 128
NH = 8
NGRAPH = 64
NCLS = 10
NTILES = 32
EPT = E // NTILES
RSTEP = 624
RSPAN = 640
BLK = 1000
GRID = N // BLK

_f32 = jnp.float32
_i32 = jnp.int32


def _mesh():
    return plsc.VectorSubcoreMesh(core_axis_name="c", subcore_axis_name="s",
                                  num_cores=2, num_subcores=16)


def _leaky_exp(a, b):
    e = a + b
    return jnp.exp(jnp.where(e > 0, e, 0.2 * e))


def _zero_m(m_sh, rows_v, rbase, cb):
    zero16 = jnp.zeros((16,), _f32)

    @pl.loop(0, cb)
    def _(e):
        for kk in range(8):
            rows_v[e, pl.ds(16 * kk, 16)] = zero16

    for i in range(RSPAN // cb):
        pltpu.sync_copy(rows_v, m_sh.at[pl.ds(rbase + i * cb, cb)])


def _export_m(m_sh, m_out, c, rbase, cb):
    for i in range(RSPAN // cb):
        pltpu.sync_copy(m_sh.at[pl.ds(rbase + i * cb, cb)],
                        m_out.at[c, pl.ds(rbase + i * cb, cb), :])


def _make_sc_layer8():
    CA = 2000
    CB = 80

    def body(h_hbm, alT_hbm, arT_hbm, src_hbm, dst_hbm,
             m_out, es_out, w_out,
             m_sh, al_v, ar_v, es_v, srcA_v, dstA_v, wa_v,
             srcB_v, dstB_v, wb0_v, wb1_v, wb2_v, wb3_v,
             wb4_v, wb5_v, wb6_v, wb7_v, rows_v, sem):
        c = lax.axis_index("c")
        s = lax.axis_index("s")
        wid = s * 2 + c
        ebase = wid * EPT
        rbase = s * RSTEP
        zero16 = jnp.zeros((16,), _f32)
        wbs = (wb0_v, wb1_v, wb2_v, wb3_v, wb4_v, wb5_v, wb6_v, wb7_v)

        _zero_m(m_sh, rows_v, rbase, CB)
        plsc.subcore_barrier()

        for g in range(NH):
            pltpu.sync_copy(alT_hbm.at[pl.ds(g * N, N)], al_v)
            pltpu.sync_copy(arT_hbm.at[pl.ds(g * N, N)], ar_v)

            @pl.loop(0, N // 16)
            def _(i):
                es_v[pl.ds(16 * i, 16)] = zero16

            @pl.loop(0, EPT // CA)
            def _(ci):
                cb = ebase + ci * CA
                pltpu.sync_copy(src_hbm.at[pl.ds(cb, CA)], srcA_v)
                pltpu.sync_copy(dst_hbm.at[pl.ds(cb, CA)], dstA_v)

                @pl.loop(0, CA // 16)
                def _(j):
                    s16 = srcA_v[pl.ds(16 * j, 16)]
                    d16 = dstA_v[pl.ds(16 * j, 16)]
                    w = _leaky_exp(plsc.load_gather(al_v, [s16]),
                                   plsc.load_gather(ar_v, [d16]))
                    plsc.addupdate_scatter(es_v, [d16], w)
                    wa_v[pl.ds(16 * j, 16)] = w

                pltpu.sync_copy(wa_v, w_out.at[pl.ds(g * E + cb, CA)])

            pltpu.sync_copy(es_v, es_out.at[pl.ds((wid * NH + g) * N, N)])

        @pl.loop(0, EPT // CB)
        def _(ci):
            cb = ebase + ci * CB
            pltpu.sync_copy(src_hbm.at[pl.ds(cb, CB)], srcB_v)
            pltpu.sync_copy(dst_hbm.at[pl.ds(cb, CB)], dstB_v)
            for g in range(NH):
                pltpu.sync_copy(w_out.at[pl.ds(g * E + cb, CB)], wbs[g])
            pltpu.async_copy(h_hbm.at[srcB_v], rows_v, sem).wait()

            @pl.loop(0, CB // 16)
            def _(q):
                for g in range(NH):
                    wq = wbs[g][pl.ds(16 * q, 16)]
                    for i in range(16):
                        e = 16 * q + i
                        rows_v[e, pl.ds(16 * g, 16)] = (
                            rows_v[e, pl.ds(16 * g, 16)] * wq[i])

            pltpu.sync_copy(rows_v, m_sh.at[dstB_v], add=True)

        plsc.subcore_barrier()
        _export_m(m_sh, m_out, c, rbase, CB)

    return pl.kernel(
        body,
        out_type=[
            jax.ShapeDtypeStruct((2, N, F), _f32),
            jax.ShapeDtypeStruct((NTILES * NH * N,), _f32),
            jax.ShapeDtypeStruct((NH * E,), _f32),
        ],
        mesh=_mesh(),
        compiler_params=pltpu.CompilerParams(needs_layout_passes=False),
        scratch_types=[
            pltpu.VMEM_SHARED((N, F), _f32),
            pltpu.VMEM((N,), _f32),
            pltpu.VMEM((N,), _f32),
            pltpu.VMEM((N,), _f32),
            pltpu.VMEM((CA,), _i32),
            pltpu.VMEM((CA,), _i32),
            pltpu.VMEM((CA,), _f32),
            pltpu.VMEM((CB,), _i32),
            pltpu.VMEM((CB,), _i32),
        ] + [pltpu.VMEM((CB,), _f32) for _ in range(NH)] + [
            pltpu.VMEM((CB, F), _f32),
            pltpu.SemaphoreType.DMA,
        ],
    )


def _make_sc_layer1():
    CB = 80

    def body(h_hbm, al_hbm, ar_hbm, src_hbm, dst_hbm, m_out, es_out,
             m_sh, al_v, ar_v, es_v, src_v, dst_v, w_v, rows_v, sem):
        c = lax.axis_index("c")
        s = lax.axis_index("s")
        wid = s * 2 + c
        ebase = wid * EPT
        rbase = s * RSTEP
        zero16 = jnp.zeros((16,), _f32)

        _zero_m(m_sh, rows_v, rbase, CB)
        plsc.subcore_barrier()

        pltpu.sync_copy(al_hbm, al_v)
        pltpu.sync_copy(ar_hbm, ar_v)

        @pl.loop(0, N // 16)
        def _(i):
            es_v[pl.ds(16 * i, 16)] = zero16

        @pl.loop(0, EPT // CB)
        def _(ci):
            cb = ebase + ci * CB
            pltpu.sync_copy(src_hbm.at[pl.ds(cb, CB)], src_v)
            pltpu.sync_copy(dst_hbm.at[pl.ds(cb, CB)], dst_v)
            pltpu.async_copy(h_hbm.at[src_v], rows_v, sem).wait()

            @pl.loop(0, CB // 16)
            def _(j):
                s16 = src_v[pl.ds(16 * j, 16)]
                d16 = dst_v[pl.ds(16 * j, 16)]
                w = _leaky_exp(plsc.load_gather(al_v, [s16]),
                               plsc.load_gather(ar_v, [d16]))
                w_v[pl.ds(16 * j, 16)] = w
                plsc.addupdate_scatter(es_v, [d16], w)

            @pl.loop(0, CB // 16)
            def _(j):
                wvec = w_v[pl.ds(16 * j, 16)]
                for i in range(16):
                    e = 16 * j + i
                    for kk in range(8):
                        rows_v[e, pl.ds(16 * kk, 16)] = (
                            rows_v[e, pl.ds(16 * kk, 16)] * wvec[i])

            pltpu.sync_copy(rows_v, m_sh.at[dst_v], add=True)

        pltpu.sync_copy(es_v, es_out.at[pl.ds(wid * N, N)])
        plsc.subcore_barrier()
        _export_m(m_sh, m_out, c, rbase, CB)

    return pl.kernel(
        body,
        out_type=[
            jax.ShapeDtypeStruct((2, N, F), _f32),
            jax.ShapeDtypeStruct((NTILES * N,), _f32),
        ],
        mesh=_mesh(),
        compiler_params=pltpu.CompilerParams(needs_layout_passes=False),
        scratch_types=[
            pltpu.VMEM_SHARED((N, F), _f32),
            pltpu.VMEM((N,), _f32),
            pltpu.VMEM((N,), _f32),
            pltpu.VMEM((N,), _f32),
            pltpu.VMEM((CB,), _i32),
            pltpu.VMEM((CB,), _i32),
            pltpu.VMEM((CB,), _f32),
            pltpu.VMEM((CB, F), _f32),
            pltpu.SemaphoreType.DMA,
        ],
    )


_sc_cache = {}


def _sc_layer(H):
    if H not in _sc_cache:
        _sc_cache[H] = _make_sc_layer8() if H == 8 else _make_sc_layer1()
    return _sc_cache[H]


def _proj_kernel(x_ref, w_ref, acat_ref, h_ref, tab_ref):
    h = jnp.dot(x_ref[...], w_ref[...], preferred_element_type=_f32)
    h_ref[...] = h
    tab_ref[...] = jnp.dot(h, acat_ref[...], preferred_element_type=_f32)


def _head_expander(H):
    if H == 1:
        return jnp.ones((1, F), _f32)
    width = F // NH
    return (lax.broadcasted_iota(_i32, (H, H * width), 0)
            == lax.broadcasted_iota(_i32, (H, H * width), 1) // width
            ).astype(_f32)


def _combine_x(H, m_ref, es_ref, al_ref, ar_ref, h_ref, b_ref):
    if H == 8:
        aa = al_ref[...][:, 0:NH] + ar_ref[...][:, NH:2 * NH]
    else:
        aa = al_ref[...] + ar_ref[...]
    wself = jnp.exp(jnp.where(aa > 0, aa, 0.2 * aa))
    nt = NTILES * H
    r = (lax.broadcasted_iota(_i32, (nt, H), 0) % H
         == lax.broadcasted_iota(_i32, (nt, H), 1)).astype(_f32)
    es = jnp.dot(es_ref[...], r, preferred_element_type=_f32)
    es = es + wself
    p = _head_expander(H)
    wx = jnp.dot(wself, p, preferred_element_type=_f32)
    esx = jnp.dot(es, p, preferred_element_type=_f32)
    m = m_ref[0] + m_ref[1] + wx * h_ref[...]
    x = m / esx + b_ref[...]
    return jnp.where(x > 0, x, jnp.exp(jnp.minimum(x, 0.0)) - 1.0)


def _make_combine_kernel(H):
    def kern(m_ref, es_ref, al_ref, ar_ref, h_ref, b_ref,
             w_ref, as_ref, ad_ref, hn_ref, aln_ref, arn_ref):
        x = _combine_x(H, m_ref, es_ref, al_ref, ar_ref, h_ref, b_ref)
        hn = jnp.dot(x, w_ref[...], preferred_element_type=_f32)
        hn_ref[...] = hn
        aln_ref[...] = jnp.dot(hn, as_ref[...], preferred_element_type=_f32)
        arn_ref[...] = jnp.dot(hn, ad_ref[...], preferred_element_type=_f32)
    return kern


def _pool_kernel(m_ref, es_ref, al_ref, ar_ref, h_ref, b_ref, batch_ref,
                 sums_ref, cnt_ref):
    x = _combine_x(1, m_ref, es_ref, al_ref, ar_ref, h_ref, b_ref)
    bb = batch_ref[...].reshape(1, BLK)
    onehot = (bb == lax.broadcasted_iota(_i32, (NGRAPH, 1), 0)).astype(_f32)

    @pl.when(pl.program_id(0) == 0)
    def _():
        sums_ref[...] = jnp.zeros_like(sums_ref)
        cnt_ref[...] = jnp.zeros_like(cnt_ref)

    sums_ref[...] += jnp.dot(onehot, x, preferred_element_type=_f32)
    cnt_ref[...] += jnp.dot(onehot, jnp.ones((BLK, F), _f32),
                            preferred_element_type=_f32)


def _mlp_kernel(sums_ref, cnt_ref, l1w_ref, l1b_ref, l2w_ref, l2b_ref, out_ref):
    pooled = sums_ref[...] / jnp.maximum(cnt_ref[...], 1.0)
    z = jnp.dot(pooled, l1w_ref[...], preferred_element_type=_f32) + l1b_ref[...]
    z = jnp.maximum(z, 0.0)
    z = jnp.dot(z, l2w_ref[...], preferred_element_type=_f32) + l2b_ref[...]
    z = z - jnp.max(z, axis=-1, keepdims=True)
    out_ref[...] = z - jnp.log(jnp.sum(jnp.exp(z), axis=-1, keepdims=True))


def _row_spec():
    return pl.BlockSpec((BLK, F), lambda i: (i, 0))


def _head_spec(H):
    return pl.BlockSpec((BLK, H), lambda i: (i, 0))


def _fixed(shape):
    return pl.BlockSpec(shape, lambda i: tuple(0 for _ in shape))


def _proj(x, w, acat):
    return pl.pallas_call(
        _proj_kernel,
        grid=(GRID,),
        in_specs=[_row_spec(), _fixed((F, F)), _fixed((F, F))],
        out_specs=[_row_spec(), _row_spec()],
        out_shape=[
            jax.ShapeDtypeStruct((N, F), _f32),
            jax.ShapeDtypeStruct((N, F), _f32),
        ],
    )(x, w, acat)


def _es_spec(H):
    return pl.BlockSpec((BLK, NTILES * H), lambda i: (i, 0))


def _es_shaped(es, H):
    return es.reshape(NTILES * H, N).T


def _combine_specs(H):
    logit_spec = _row_spec() if H == 8 else _head_spec(H)
    return [
        pl.BlockSpec((2, BLK, F), lambda i: (0, i, 0)),
        _es_spec(H),
        logit_spec,
        logit_spec,
        _row_spec(),
        _fixed((1, F)),
    ]


def _combine(H, H2, m, es, al, ar, h, b, w, a_s, a_d):
    return pl.pallas_call(
        _make_combine_kernel(H),
        grid=(GRID,),
        in_specs=_combine_specs(H) + [
            _fixed((F, F)), _fixed((F, H2)), _fixed((F, H2))],
        out_specs=[_row_spec(), _head_spec(H2), _head_spec(H2)],
        out_shape=[
            jax.ShapeDtypeStruct((N, F), _f32),
            jax.ShapeDtypeStruct((N, H2), _f32),
            jax.ShapeDtypeStruct((N, H2), _f32),
        ],
    )(m, _es_shaped(es, H), al, ar, h, b, w, a_s, a_d)


def _pool(m, es, al, ar, h, b, batch3):
    return pl.pallas_call(
        _pool_kernel,
        grid=(GRID,),
        in_specs=_combine_specs(1) + [
            pl.BlockSpec((1, 1, BLK), lambda i: (i, 0, 0))],
        out_specs=[_fixed((NGRAPH, F)), _fixed((NGRAPH, F))],
        out_shape=[
            jax.ShapeDtypeStruct((NGRAPH, F), _f32),
            jax.ShapeDtypeStruct((NGRAPH, F), _f32),
        ],
    )(m, _es_shaped(es, 1), al, ar, h, b, batch3)


def _expand_a(a, heads):
    flat = a.reshape(-1)
    if heads == 8:
        col = jnp.arange(F) // 16
        row = jnp.arange(NH)
        return (col[:, None] == row[None, :]).astype(_f32) * flat[:, None]
    return flat[:, None]


def kernel(x, edge_index, batch, W1, a_s1, a_d1, b1, W2, a_s2, a_d2, b2,
           W3, a_s3, a_d3, b3, lin1_w, lin1_b, lin2_w, lin2_b):
    src = edge_index[0]
    dst = edge_index[1]
    batch3 = batch.reshape(GRID, 1, BLK)

    acat1 = jnp.concatenate(
        [_expand_a(a_s1, 8), _expand_a(a_d1, 8), jnp.zeros((F, F - 16), _f32)],
        axis=1)
    as2, ad2 = _expand_a(a_s2, 1), _expand_a(a_d2, 1)
    as3, ad3 = _expand_a(a_s3, 1), _expand_a(a_d3, 1)
    b1r, b2r, b3r = b1.reshape(1, F), b2.reshape(1, F), b3.reshape(1, F)

    h1, tab1 = _proj(x, W1, acat1)
    alT1 = tab1[:, 0:NH].T.reshape(-1)
    arT1 = tab1[:, NH:2 * NH].T.reshape(-1)
    m1, es1, _ = _sc_layer(8)(h1, alT1, arT1, src, dst)
    h2, al2, ar2 = _combine(8, 1, m1, es1, tab1, tab1, h1, b1r, W2, as2, ad2)
    m2, es2 = _sc_layer(1)(h2, al2.reshape(-1), ar2.reshape(-1), src, dst)
    h3, al3, ar3 = _combine(1, 1, m2, es2, al2, ar2, h2, b2r, W3, as3, ad3)
    m3, es3 = _sc_layer(1)(h3, al3.reshape(-1), ar3.reshape(-1), src, dst)
    sums, cnt = _pool(m3, es3, al3, ar3, h3, b3r, batch3)
    out = pl.pallas_call(
        _mlp_kernel,
        out_shape=jax.ShapeDtypeStruct((NGRAPH, NCLS), _f32),
    )(sums, cnt, lin1_w, lin1_b.reshape(1, -1), lin2_w, lin2_b.reshape(1, -1))
    return out


# speedup vs baseline: 36.2018x; 1.0844x over previous
import jax
import jax.numpy as jnp
from jax import lax
from jax.experimental import pallas as pl
from jax.experimental.pallas import tpu as pltpu
from jax.experimental.pallas import tpu_sc as plsc

N = 10000
E = 320000
F = 128
NH = 8
NGRAPH = 64
NCLS = 10
NTILES = 32
EPT = E // NTILES
RSTEP = 624
RSPAN = 640
BLK = 1000
GRID = N // BLK

_f32 = jnp.float32
_i32 = jnp.int32


def _mesh():
    return plsc.VectorSubcoreMesh(core_axis_name="c", subcore_axis_name="s",
                                  num_cores=2, num_subcores=16)


def _leaky_exp(a, b):
    e = a + b
    return jnp.exp(jnp.where(e > 0, e, 0.2 * e))


def _zero_m(m_sh, rows_v, rbase, cb):
    zero16 = jnp.zeros((16,), _f32)

    @pl.loop(0, cb)
    def _(e):
        for kk in range(8):
            rows_v[e, pl.ds(16 * kk, 16)] = zero16

    for i in range(RSPAN // cb):
        pltpu.sync_copy(rows_v, m_sh.at[pl.ds(rbase + i * cb, cb)])


def _export_m(m_sh, m_out, c, rbase, cb):
    for i in range(RSPAN // cb):
        pltpu.sync_copy(m_sh.at[pl.ds(rbase + i * cb, cb)],
                        m_out.at[c, pl.ds(rbase + i * cb, cb), :])


def _make_sc_layer8():
    CA = 2000
    CB = 80

    def body(h_hbm, alT_hbm, arT_hbm, src_hbm, dst_hbm,
             m_out, es_out, w_out,
             m_sh, al_v, ar_v, es_v, srcA_v, dstA_v, wa_v,
             srcB_v, dstB_v, wb0_v, wb1_v, wb2_v, wb3_v,
             wb4_v, wb5_v, wb6_v, wb7_v, rows_v, sem):
        c = lax.axis_index("c")
        s = lax.axis_index("s")
        wid = s * 2 + c
        ebase = wid * EPT
        rbase = s * RSTEP
        zero16 = jnp.zeros((16,), _f32)
        wbs = (wb0_v, wb1_v, wb2_v, wb3_v, wb4_v, wb5_v, wb6_v, wb7_v)

        _zero_m(m_sh, rows_v, rbase, CB)
        plsc.subcore_barrier()

        for g in range(NH):
            pltpu.sync_copy(alT_hbm.at[pl.ds(g * N, N)], al_v)
            pltpu.sync_copy(arT_hbm.at[pl.ds(g * N, N)], ar_v)

            @pl.loop(0, N // 16)
            def _(i):
                es_v[pl.ds(16 * i, 16)] = zero16

            @pl.loop(0, EPT // CA)
            def _(ci):
                cb = ebase + ci * CA
                pltpu.sync_copy(src_hbm.at[pl.ds(cb, CA)], srcA_v)
                pltpu.sync_copy(dst_hbm.at[pl.ds(cb, CA)], dstA_v)

                @pl.loop(0, CA // 16)
                def _(j):
                    s16 = srcA_v[pl.ds(16 * j, 16)]
                    d16 = dstA_v[pl.ds(16 * j, 16)]
                    w = _leaky_exp(plsc.load_gather(al_v, [s16]),
                                   plsc.load_gather(ar_v, [d16]))
                    plsc.addupdate_scatter(es_v, [d16], w)
                    wa_v[pl.ds(16 * j, 16)] = w

                pltpu.sync_copy(wa_v, w_out.at[pl.ds(g * E + cb, CA)])

            pltpu.sync_copy(es_v, es_out.at[pl.ds((wid * NH + g) * N, N)])

        @pl.loop(0, EPT // CB)
        def _(ci):
            cb = ebase + ci * CB
            pltpu.sync_copy(src_hbm.at[pl.ds(cb, CB)], srcB_v)
            pltpu.sync_copy(dst_hbm.at[pl.ds(cb, CB)], dstB_v)
            gather = pltpu.async_copy(h_hbm.at[srcB_v], rows_v, sem)
            for g in range(NH):
                pltpu.sync_copy(w_out.at[pl.ds(g * E + cb, CB)], wbs[g])
            gather.wait()

            @pl.loop(0, CB // 16)
            def _(q):
                for g in range(NH):
                    wq = wbs[g][pl.ds(16 * q, 16)]
                    for i in range(16):
                        e = 16 * q + i
                        rows_v[e, pl.ds(16 * g, 16)] = (
                            rows_v[e, pl.ds(16 * g, 16)] * wq[i])

            pltpu.sync_copy(rows_v, m_sh.at[dstB_v], add=True)

        plsc.subcore_barrier()
        _export_m(m_sh, m_out, c, rbase, CB)

    return pl.kernel(
        body,
        out_type=[
            jax.ShapeDtypeStruct((2, N, F), _f32),
            jax.ShapeDtypeStruct((NTILES * NH * N,), _f32),
            jax.ShapeDtypeStruct((NH * E,), _f32),
        ],
        mesh=_mesh(),
        compiler_params=pltpu.CompilerParams(needs_layout_passes=False),
        scratch_types=[
            pltpu.VMEM_SHARED((N, F), _f32),
            pltpu.VMEM((N,), _f32),
            pltpu.VMEM((N,), _f32),
            pltpu.VMEM((N,), _f32),
            pltpu.VMEM((CA,), _i32),
            pltpu.VMEM((CA,), _i32),
            pltpu.VMEM((CA,), _f32),
            pltpu.VMEM((CB,), _i32),
            pltpu.VMEM((CB,), _i32),
        ] + [pltpu.VMEM((CB,), _f32) for _ in range(NH)] + [
            pltpu.VMEM((CB, F), _f32),
            pltpu.SemaphoreType.DMA,
        ],
    )


def _make_sc_layer1():
    CB = 80

    def body(h_hbm, al_hbm, ar_hbm, src_hbm, dst_hbm, m_out, es_out,
             m_sh, al_v, ar_v, es_v, src_v, dst_v, w_v, rows_v, sem):
        c = lax.axis_index("c")
        s = lax.axis_index("s")
        wid = s * 2 + c
        ebase = wid * EPT
        rbase = s * RSTEP
        zero16 = jnp.zeros((16,), _f32)

        _zero_m(m_sh, rows_v, rbase, CB)
        plsc.subcore_barrier()

        pltpu.sync_copy(al_hbm, al_v)
        pltpu.sync_copy(ar_hbm, ar_v)

        @pl.loop(0, N // 16)
        def _(i):
            es_v[pl.ds(16 * i, 16)] = zero16

        @pl.loop(0, EPT // CB)
        def _(ci):
            cb = ebase + ci * CB
            pltpu.sync_copy(src_hbm.at[pl.ds(cb, CB)], src_v)
            pltpu.sync_copy(dst_hbm.at[pl.ds(cb, CB)], dst_v)
            gather = pltpu.async_copy(h_hbm.at[src_v], rows_v, sem)

            @pl.loop(0, CB // 16)
            def _(j):
                s16 = src_v[pl.ds(16 * j, 16)]
                d16 = dst_v[pl.ds(16 * j, 16)]
                w = _leaky_exp(plsc.load_gather(al_v, [s16]),
                               plsc.load_gather(ar_v, [d16]))
                w_v[pl.ds(16 * j, 16)] = w
                plsc.addupdate_scatter(es_v, [d16], w)

            gather.wait()

            @pl.loop(0, CB // 16)
            def _(j):
                wvec = w_v[pl.ds(16 * j, 16)]
                for i in range(16):
                    e = 16 * j + i
                    for kk in range(8):
                        rows_v[e, pl.ds(16 * kk, 16)] = (
                            rows_v[e, pl.ds(16 * kk, 16)] * wvec[i])

            pltpu.sync_copy(rows_v, m_sh.at[dst_v], add=True)

        pltpu.sync_copy(es_v, es_out.at[pl.ds(wid * N, N)])
        plsc.subcore_barrier()
        _export_m(m_sh, m_out, c, rbase, CB)

    return pl.kernel(
        body,
        out_type=[
            jax.ShapeDtypeStruct((2, N, F), _f32),
            jax.ShapeDtypeStruct((NTILES * N,), _f32),
        ],
        mesh=_mesh(),
        compiler_params=pltpu.CompilerParams(needs_layout_passes=False),
        scratch_types=[
            pltpu.VMEM_SHARED((N, F), _f32),
            pltpu.VMEM((N,), _f32),
            pltpu.VMEM((N,), _f32),
            pltpu.VMEM((N,), _f32),
            pltpu.VMEM((CB,), _i32),
            pltpu.VMEM((CB,), _i32),
            pltpu.VMEM((CB,), _f32),
            pltpu.VMEM((CB, F), _f32),
            pltpu.SemaphoreType.DMA,
        ],
    )


_sc_cache = {}


def _sc_layer(H):
    if H not in _sc_cache:
        _sc_cache[H] = _make_sc_layer8() if H == 8 else _make_sc_layer1()
    return _sc_cache[H]


def _proj_kernel(x_ref, w_ref, acat_ref, h_ref, tab_ref):
    h = jnp.dot(x_ref[...], w_ref[...], preferred_element_type=_f32)
    h_ref[...] = h
    tab_ref[...] = jnp.dot(h, acat_ref[...], preferred_element_type=_f32)


def _head_expander(H):
    if H == 1:
        return jnp.ones((1, F), _f32)
    width = F // NH
    return (lax.broadcasted_iota(_i32, (H, H * width), 0)
            == lax.broadcasted_iota(_i32, (H, H * width), 1) // width
            ).astype(_f32)


def _combine_x(H, m_ref, es_ref, al_ref, ar_ref, h_ref, b_ref):
    if H == 8:
        aa = al_ref[...][:, 0:NH] + ar_ref[...][:, NH:2 * NH]
    else:
        aa = al_ref[...] + ar_ref[...]
    wself = jnp.exp(jnp.where(aa > 0, aa, 0.2 * aa))
    nt = NTILES * H
    r = (lax.broadcasted_iota(_i32, (nt, H), 0) % H
         == lax.broadcasted_iota(_i32, (nt, H), 1)).astype(_f32)
    es = jnp.dot(es_ref[...], r, preferred_element_type=_f32)
    es = es + wself
    p = _head_expander(H)
    wx = jnp.dot(wself, p, preferred_element_type=_f32)
    esx = jnp.dot(es, p, preferred_element_type=_f32)
    m = m_ref[0] + m_ref[1] + wx * h_ref[...]
    x = m / esx + b_ref[...]
    return jnp.where(x > 0, x, jnp.exp(jnp.minimum(x, 0.0)) - 1.0)


def _make_combine_kernel(H):
    def kern(m_ref, es_ref, al_ref, ar_ref, h_ref, b_ref,
             w_ref, as_ref, ad_ref, hn_ref, aln_ref, arn_ref):
        x = _combine_x(H, m_ref, es_ref, al_ref, ar_ref, h_ref, b_ref)
        hn = jnp.dot(x, w_ref[...], preferred_element_type=_f32)
        hn_ref[...] = hn
        aln_ref[...] = jnp.dot(hn, as_ref[...], preferred_element_type=_f32)
        arn_ref[...] = jnp.dot(hn, ad_ref[...], preferred_element_type=_f32)
    return kern


def _pool_kernel(m_ref, es_ref, al_ref, ar_ref, h_ref, b_ref, batch_ref,
                 sums_ref, cnt_ref):
    x = _combine_x(1, m_ref, es_ref, al_ref, ar_ref, h_ref, b_ref)
    bb = batch_ref[...].reshape(1, BLK)
    onehot = (bb == lax.broadcasted_iota(_i32, (NGRAPH, 1), 0)).astype(_f32)

    @pl.when(pl.program_id(0) == 0)
    def _():
        sums_ref[...] = jnp.zeros_like(sums_ref)
        cnt_ref[...] = jnp.zeros_like(cnt_ref)

    sums_ref[...] += jnp.dot(onehot, x, preferred_element_type=_f32)
    cnt_ref[...] += jnp.dot(onehot, jnp.ones((BLK, F), _f32),
                            preferred_element_type=_f32)


def _mlp_kernel(sums_ref, cnt_ref, l1w_ref, l1b_ref, l2w_ref, l2b_ref, out_ref):
    pooled = sums_ref[...] / jnp.maximum(cnt_ref[...], 1.0)
    z = jnp.dot(pooled, l1w_ref[...], preferred_element_type=_f32) + l1b_ref[...]
    z = jnp.maximum(z, 0.0)
    z = jnp.dot(z, l2w_ref[...], preferred_element_type=_f32) + l2b_ref[...]
    z = z - jnp.max(z, axis=-1, keepdims=True)
    out_ref[...] = z - jnp.log(jnp.sum(jnp.exp(z), axis=-1, keepdims=True))


def _row_spec():
    return pl.BlockSpec((BLK, F), lambda i: (i, 0))


def _head_spec(H):
    return pl.BlockSpec((BLK, H), lambda i: (i, 0))


def _fixed(shape):
    return pl.BlockSpec(shape, lambda i: tuple(0 for _ in shape))


def _proj(x, w, acat):
    return pl.pallas_call(
        _proj_kernel,
        grid=(GRID,),
        in_specs=[_row_spec(), _fixed((F, F)), _fixed((F, F))],
        out_specs=[_row_spec(), _row_spec()],
        out_shape=[
            jax.ShapeDtypeStruct((N, F), _f32),
            jax.ShapeDtypeStruct((N, F), _f32),
        ],
    )(x, w, acat)


def _es_spec(H):
    return pl.BlockSpec((BLK, NTILES * H), lambda i: (i, 0))


def _es_shaped(es, H):
    return es.reshape(NTILES * H, N).T


def _combine_specs(H):
    logit_spec = _row_spec() if H == 8 else _head_spec(H)
    return [
        pl.BlockSpec((2, BLK, F), lambda i: (0, i, 0)),
        _es_spec(H),
        logit_spec,
        logit_spec,
        _row_spec(),
        _fixed((1, F)),
    ]


def _combine(H, H2, m, es, al, ar, h, b, w, a_s, a_d):
    return pl.pallas_call(
        _make_combine_kernel(H),
        grid=(GRID,),
        in_specs=_combine_specs(H) + [
            _fixed((F, F)), _fixed((F, H2)), _fixed((F, H2))],
        out_specs=[_row_spec(), _head_spec(H2), _head_spec(H2)],
        out_shape=[
            jax.ShapeDtypeStruct((N, F), _f32),
            jax.ShapeDtypeStruct((N, H2), _f32),
            jax.ShapeDtypeStruct((N, H2), _f32),
        ],
    )(m, _es_shaped(es, H), al, ar, h, b, w, a_s, a_d)


def _pool(m, es, al, ar, h, b, batch3):
    return pl.pallas_call(
        _pool_kernel,
        grid=(GRID,),
        in_specs=_combine_specs(1) + [
            pl.BlockSpec((1, 1, BLK), lambda i: (i, 0, 0))],
        out_specs=[_fixed((NGRAPH, F)), _fixed((NGRAPH, F))],
        out_shape=[
            jax.ShapeDtypeStruct((NGRAPH, F), _f32),
            jax.ShapeDtypeStruct((NGRAPH, F), _f32),
        ],
    )(m, _es_shaped(es, 1), al, ar, h, b, batch3)


def _expand_a(a, heads):
    flat = a.reshape(-1)
    if heads == 8:
        col = jnp.arange(F) // 16
        row = jnp.arange(NH)
        return (col[:, None] == row[None, :]).astype(_f32) * flat[:, None]
    return flat[:, None]


def kernel(x, edge_index, batch, W1, a_s1, a_d1, b1, W2, a_s2, a_d2, b2,
           W3, a_s3, a_d3, b3, lin1_w, lin1_b, lin2_w, lin2_b):
    src = edge_index[0]
    dst = edge_index[1]
    batch3 = batch.reshape(GRID, 1, BLK)

    acat1 = jnp.concatenate(
        [_expand_a(a_s1, 8), _expand_a(a_d1, 8), jnp.zeros((F, F - 16), _f32)],
        axis=1)
    as2, ad2 = _expand_a(a_s2, 1), _expand_a(a_d2, 1)
    as3, ad3 = _expand_a(a_s3, 1), _expand_a(a_d3, 1)
    b1r, b2r, b3r = b1.reshape(1, F), b2.reshape(1, F), b3.reshape(1, F)

    h1, tab1 = _proj(x, W1, acat1)
    alT1 = tab1[:, 0:NH].T.reshape(-1)
    arT1 = tab1[:, NH:2 * NH].T.reshape(-1)
    m1, es1, _ = _sc_layer(8)(h1, alT1, arT1, src, dst)
    h2, al2, ar2 = _combine(8, 1, m1, es1, tab1, tab1, h1, b1r, W2, as2, ad2)
    m2, es2 = _sc_layer(1)(h2, al2.reshape(-1), ar2.reshape(-1), src, dst)
    h3, al3, ar3 = _combine(1, 1, m2, es2, al2, ar2, h2, b2r, W3, as3, ad3)
    m3, es3 = _sc_layer(1)(h3, al3.reshape(-1), ar3.reshape(-1), src, dst)
    sums, cnt = _pool(m3, es3, al3, ar3, h3, b3r, batch3)
    out = pl.pallas_call(
        _mlp_kernel,
        out_shape=jax.ShapeDtypeStruct((NGRAPH, NCLS), _f32),
    )(sums, cnt, lin1_w, lin1_b.reshape(1, -1), lin2_w, lin2_b.reshape(1, -1))
    return out
